# Initial kernel scaffold; baseline (speedup 1.0000x reference)
import jax
import jax.numpy as jnp
from jax.experimental import pallas as pl


def kernel(user_id, item_id, price, user_age, item_table, price_table, bn1_gamma, bn1_beta, W1, b1, bn2_gamma, bn2_beta, W2, b2):
    raise NotImplementedError("write your pallas kernel here")


# speedup vs baseline: 2.7986x; 2.7986x over previous
import functools
import math

import jax
import jax.numpy as jnp
from jax import lax
from jax.experimental import pallas as pl
from jax.experimental.pallas import tpu as pltpu
from jax.experimental.pallas import tpu_sc as plsc

_BOUNDS = tuple(float(b) for b in range(1, 100, 10))
_INV_SQRT = 1.0 / math.sqrt(1.0 + 1e-3)
_ITEM_PAD = 128
_N_BUCKETS = 11


def _table_body(pt_ref, it_ref, g1_ref, be1_ref, w1_ref, b1_ref, g2_ref,
                be2_ref, w2_ref, b2_ref, out_ref):
    w1 = w1_ref[...]
    g1 = g1_ref[...] * _INV_SQRT
    a = jnp.dot(pt_ref[...] * g1[:, :32], w1[:32, :],
                preferred_element_type=jnp.float32)
    c = jnp.dot(it_ref[...] * g1[:, 32:], w1[32:, :],
                preferred_element_type=jnp.float32)
    d1 = jnp.dot(be1_ref[...], w1, preferred_element_type=jnp.float32) \
        + b1_ref[...]
    g2 = g2_ref[...] * _INV_SQRT
    w2 = w2_ref[...]
    d2 = jnp.dot(be2_ref[...], w2, preferred_element_type=jnp.float32) \
        + b2_ref[...]
    for b in range(_N_BUCKETS):
        h = jnp.maximum(a[b:b + 1, :] + c + d1, 0.0)
        t = jnp.maximum(
            jnp.dot(h * g2, w2, preferred_element_type=jnp.float32) + d2,
            0.0)
        out_ref[b * _ITEM_PAD:(b + 1) * _ITEM_PAD, :] = t


def _build_table(ptp, itp, g1, be1, w1, b1, g2, be2, w2, b2):
    return pl.pallas_call(
        _table_body,
        out_shape=jax.ShapeDtypeStruct((_N_BUCKETS * _ITEM_PAD, 16),
                                       jnp.float32),
    )(ptp, itp, g1, be1, w1, b1, g2, be2, w2, b2)


@functools.cache
def _make_sc_lookup(batch):
    info = plsc.get_sparse_core_info()
    n_workers = info.num_cores * info.num_subcores
    bpw = batch // n_workers
    n_table = _N_BUCKETS * _ITEM_PAD * 16
    mesh = plsc.VectorSubcoreMesh(core_axis_name="c", subcore_axis_name="s")

    @functools.partial(
        pl.kernel,
        mesh=mesh,
        out_type=jax.ShapeDtypeStruct((batch, 16), jnp.float32),
        compiler_params=pltpu.CompilerParams(needs_layout_passes=False),
        scratch_types=[
            pltpu.VMEM((bpw,), jnp.float32),
            pltpu.VMEM((bpw,), jnp.int32),
            pltpu.VMEM((n_table,), jnp.float32),
            pltpu.VMEM((bpw, 16), jnp.float32),
            pltpu.SemaphoreType.DMA,
        ],
    )
    def sc_lookup(price_hbm, item_hbm, tflat_hbm, out_hbm,
                  price_v, item_v, table_v, rows_v, sem):
        wid = lax.axis_index("s") * info.num_cores + lax.axis_index("c")
        base = wid * bpw
        pltpu.sync_copy(tflat_hbm, table_v)
        pltpu.sync_copy(price_hbm.at[pl.ds(base, bpw)], price_v)
        pltpu.sync_copy(item_hbm.at[pl.ds(base, bpw)], item_v)
        iota = lax.iota(jnp.int32, 16)
        ones = jnp.full((16,), 1, jnp.int32)
        zeros = jnp.full((16,), 0, jnp.int32)
        for g in range(bpw // 16):
            p = price_v[pl.ds(g * 16, 16)]
            bucket = zeros
            for bound in _BOUNDS:
                bucket = bucket + jnp.where(
                    p >= jnp.full((16,), bound, jnp.float32), ones, zeros)
            combo = bucket * _ITEM_PAD + item_v[pl.ds(g * 16, 16)]
            fidx = combo * 16
            rid = iota + g * 16
            for col in range(16):
                colv = jnp.full((16,), col, jnp.int32)
                vals = plsc.load_gather(table_v, [fidx + colv])
                plsc.store_scatter(rows_v, [rid, colv], vals)
        pltpu.sync_copy(rows_v, out_hbm.at[pl.ds(base, bpw)])

    return sc_lookup


def kernel(user_id, item_id, price, user_age, item_table, price_table,
           bn1_gamma, bn1_beta, W1, b1, bn2_gamma, bn2_beta, W2, b2):
    itp = jnp.zeros((_ITEM_PAD, 32), jnp.float32).at[:item_table.shape[0]
                                                     ].set(item_table)
    ptp = jnp.zeros((16, 32), jnp.float32).at[:price_table.shape[0]
                                              ].set(price_table)
    table = _build_table(ptp, itp, bn1_gamma.reshape(1, 64),
                         bn1_beta.reshape(1, 64), W1, b1.reshape(1, 32),
                         bn2_gamma.reshape(1, 32), bn2_beta.reshape(1, 32),
                         W2, b2.reshape(1, 16))
    return _make_sc_lookup(price.shape[0])(price, item_id,
                                           table.reshape(-1))


# speedup vs baseline: 2.9557x; 1.0561x over previous
import functools
import math

import jax
import jax.numpy as jnp
from jax import lax
from jax.experimental import pallas as pl
from jax.experimental.pallas import tpu as pltpu
from jax.experimental.pallas import tpu_sc as plsc

_BOUNDS = tuple(float(b) for b in range(1, 100, 10))
_INV_SQRT = 1.0 / math.sqrt(1.0 + 1e-3)
_ITEM_PAD = 128
_N_BUCKETS = 11


def _table_body(pt_ref, it_ref, g1_ref, be1_ref, w1_ref, b1_ref, g2_ref,
                be2_ref, w2_ref, b2_ref, out_ref):
    n_items = it_ref.shape[0]
    w1 = w1_ref[...]
    g1 = g1_ref[...] * _INV_SQRT
    a = jnp.dot(pt_ref[...] * g1[:, :32], w1[:32, :],
                preferred_element_type=jnp.float32)
    c = jnp.dot(it_ref[...] * g1[:, 32:], w1[32:, :],
                preferred_element_type=jnp.float32)
    d1 = jnp.dot(be1_ref[...], w1, preferred_element_type=jnp.float32) \
        + b1_ref[...]
    g2 = g2_ref[...] * _INV_SQRT
    w2 = w2_ref[...]
    d2 = jnp.dot(be2_ref[...], w2, preferred_element_type=jnp.float32) \
        + b2_ref[...]
    for b in range(_N_BUCKETS):
        h = jnp.maximum(a[b:b + 1, :] + c + d1, 0.0)
        t = jnp.maximum(
            jnp.dot(h * g2, w2, preferred_element_type=jnp.float32) + d2,
            0.0)
        out_ref[b * _ITEM_PAD:b * _ITEM_PAD + n_items, :] = t


def _build_table(pt, it, g1, be1, w1, b1, g2, be2, w2, b2):
    return pl.pallas_call(
        _table_body,
        out_shape=jax.ShapeDtypeStruct((_N_BUCKETS * _ITEM_PAD, 16),
                                       jnp.float32),
    )(pt, it, g1, be1, w1, b1, g2, be2, w2, b2)


@functools.cache
def _make_sc_lookup(batch):
    info = plsc.get_sparse_core_info()
    n_workers = info.num_cores * info.num_subcores
    bpw = batch // n_workers
    n_rows = _N_BUCKETS * _ITEM_PAD
    mesh = plsc.VectorSubcoreMesh(core_axis_name="c", subcore_axis_name="s")

    @functools.partial(
        pl.kernel,
        mesh=mesh,
        out_type=jax.ShapeDtypeStruct((batch, 16), jnp.float32),
        compiler_params=pltpu.CompilerParams(needs_layout_passes=False),
        scratch_types=[
            pltpu.VMEM((bpw,), jnp.float32),
            pltpu.VMEM((bpw,), jnp.int32),
            pltpu.VMEM((bpw,), jnp.int32),
            pltpu.VMEM((n_rows * 16,), jnp.float32),
            pltpu.VMEM((bpw, 16), jnp.float32),
            pltpu.SemaphoreType.DMA,
            pltpu.SemaphoreType.DMA,
        ],
    )
    def sc_lookup(price_hbm, item_hbm, table_hbm, out_hbm,
                  price_v, item_v, combo_v, table_v, rows_v, sem, tsem):
        wid = lax.axis_index("s") * info.num_cores + lax.axis_index("c")
        base = wid * bpw
        tcopy = pltpu.async_copy(table_hbm, table_v, tsem)
        pcopy = pltpu.async_copy(price_hbm.at[pl.ds(base, bpw)], price_v, sem)
        icopy = pltpu.async_copy(item_hbm.at[pl.ds(base, bpw)], item_v, sem)
        pcopy.wait()
        icopy.wait()
        iota = lax.iota(jnp.int32, 16)
        ones = jnp.full((16,), 1, jnp.int32)
        zeros = jnp.full((16,), 0, jnp.int32)
        for g in range(bpw // 16):
            p = price_v[pl.ds(g * 16, 16)]
            bucket = zeros
            for bound in _BOUNDS:
                bucket = bucket + jnp.where(
                    p >= jnp.full((16,), bound, jnp.float32), ones, zeros)
            combo_v[pl.ds(g * 16, 16)] = (bucket * _ITEM_PAD
                                          + item_v[pl.ds(g * 16, 16)])
        tcopy.wait()
        for g in range(bpw // 16):
            combo = combo_v[pl.ds(g * 16, 16)]
            rid = iota + g * 16
            for col in range(16):
                colv = jnp.full((16,), col, jnp.int32)
                vals = plsc.load_gather(table_v, [combo * 16 + colv])
                plsc.store_scatter(rows_v, [rid, colv], vals)
        pltpu.sync_copy(rows_v, out_hbm.at[pl.ds(base, bpw)])

    return sc_lookup


def kernel(user_id, item_id, price, user_age, item_table, price_table,
           bn1_gamma, bn1_beta, W1, b1, bn2_gamma, bn2_beta, W2, b2):
    table = _build_table(price_table, item_table, bn1_gamma.reshape(1, 64),
                         bn1_beta.reshape(1, 64), W1, b1.reshape(1, 32),
                         bn2_gamma.reshape(1, 32), bn2_beta.reshape(1, 32),
                         W2, b2.reshape(1, 16))
    return _make_sc_lookup(price.shape[0])(price, item_id,
                                           table.reshape(-1))


# speedup vs baseline: 2.9644x; 1.0029x over previous
import functools
import math

import jax
import jax.numpy as jnp
from jax import lax
from jax.experimental import pallas as pl
from jax.experimental.pallas import tpu as pltpu
from jax.experimental.pallas import tpu_sc as plsc

_BOUNDS = tuple(float(b) for b in range(1, 100, 10))
_INV_SQRT = 1.0 / math.sqrt(1.0 + 1e-3)
_ITEM_PAD = 128
_N_BUCKETS = 11


def _table_body(pt_ref, it_ref, g1_ref, be1_ref, w1_ref, b1_ref, g2_ref,
                be2_ref, w2_ref, b2_ref, out_ref):
    n_items = it_ref.shape[0]
    w1 = w1_ref[...]
    g1 = g1_ref[...] * _INV_SQRT
    a = jnp.dot(pt_ref[...] * g1[:, :32], w1[:32, :],
                preferred_element_type=jnp.float32)
    c = jnp.dot(it_ref[...] * g1[:, 32:], w1[32:, :],
                preferred_element_type=jnp.float32)
    d1 = jnp.dot(be1_ref[...], w1, preferred_element_type=jnp.float32) \
        + b1_ref[...]
    g2 = g2_ref[...] * _INV_SQRT
    w2 = w2_ref[...]
    d2 = jnp.dot(be2_ref[...], w2, preferred_element_type=jnp.float32) \
        + b2_ref[...]
    for b in range(_N_BUCKETS):
        h = jnp.maximum(a[b:b + 1, :] + c + d1, 0.0)
        t = jnp.maximum(
            jnp.dot(h * g2, w2, preferred_element_type=jnp.float32) + d2,
            0.0)
        out_ref[b * _ITEM_PAD:b * _ITEM_PAD + n_items, :] = t


def _build_table(pt, it, g1, be1, w1, b1, g2, be2, w2, b2):
    return pl.pallas_call(
        _table_body,
        out_shape=jax.ShapeDtypeStruct((_N_BUCKETS * _ITEM_PAD, 16),
                                       jnp.float32),
    )(pt, it, g1, be1, w1, b1, g2, be2, w2, b2)


@functools.cache
def _make_sc_lookup(batch):
    info = plsc.get_sparse_core_info()
    n_workers = info.num_cores * info.num_subcores
    bpw = batch // n_workers
    n_rows = _N_BUCKETS * _ITEM_PAD
    mesh = plsc.VectorSubcoreMesh(core_axis_name="c", subcore_axis_name="s")

    @functools.partial(
        pl.kernel,
        mesh=mesh,
        out_type=jax.ShapeDtypeStruct((batch, 16), jnp.float32),
        compiler_params=pltpu.CompilerParams(needs_layout_passes=False,
                                             use_tc_tiling_on_sc=True),
        scratch_types=[
            pltpu.VMEM((bpw,), jnp.float32),
            pltpu.VMEM((bpw,), jnp.int32),
            pltpu.VMEM((bpw,), jnp.int32),
            pltpu.VMEM((n_rows * 16,), jnp.float32),
            pltpu.VMEM((bpw, 16), jnp.float32),
            pltpu.SemaphoreType.DMA,
            pltpu.SemaphoreType.DMA,
        ],
    )
    def sc_lookup(price_hbm, item_hbm, table_hbm, out_hbm,
                  price_v, item_v, combo_v, table_v, rows_v, sem, tsem):
        wid = lax.axis_index("s") * info.num_cores + lax.axis_index("c")
        base = wid * bpw
        tcopy = pltpu.async_copy(table_hbm, table_v, tsem)
        pcopy = pltpu.async_copy(price_hbm.at[pl.ds(base, bpw)], price_v, sem)
        icopy = pltpu.async_copy(item_hbm.at[pl.ds(base, bpw)], item_v, sem)
        pcopy.wait()
        icopy.wait()
        iota = lax.iota(jnp.int32, 16)
        ones = jnp.full((16,), 1, jnp.int32)
        zeros = jnp.full((16,), 0, jnp.int32)
        for g in range(bpw // 16):
            p = price_v[pl.ds(g * 16, 16)]
            bucket = zeros
            for bound in _BOUNDS:
                bucket = bucket + jnp.where(
                    p >= jnp.full((16,), bound, jnp.float32), ones, zeros)
            combo_v[pl.ds(g * 16, 16)] = (bucket * _ITEM_PAD
                                          + item_v[pl.ds(g * 16, 16)])
        tcopy.wait()
        for g in range(bpw // 16):
            combo = combo_v[pl.ds(g * 16, 16)]
            rid = iota + g * 16
            for col in range(16):
                colv = jnp.full((16,), col, jnp.int32)
                vals = plsc.load_gather(table_v, [combo * 16 + colv])
                plsc.store_scatter(rows_v, [rid, colv], vals)
        pltpu.sync_copy(rows_v, out_hbm.at[pl.ds(base, bpw)])

    return sc_lookup


def kernel(user_id, item_id, price, user_age, item_table, price_table,
           bn1_gamma, bn1_beta, W1, b1, bn2_gamma, bn2_beta, W2, b2):
    table = _build_table(price_table, item_table, bn1_gamma.reshape(1, 64),
                         bn1_beta.reshape(1, 64), W1, b1.reshape(1, 32),
                         bn2_gamma.reshape(1, 32), bn2_beta.reshape(1, 32),
                         W2, b2.reshape(1, 16))
    return _make_sc_lookup(price.shape[0])(price, item_id,
                                           table.reshape(-1))
